# baseline (device time: 111795 ns/iter reference)
import jax
import jax.numpy as jnp
from jax import lax
from jax.experimental import pallas as pl
from jax.experimental.pallas import tpu as pltpu

S = 1024
D = 2048
DC_SH = 128
H = 16
DH = 128
DR = 32
SCALE = (DH + DR) ** -0.5
S4 = S // 4
HG = 4
NG = H // HG
GW = HG * DH

F32 = jnp.float32


def _dot(a, b):
    return jnp.dot(a, b, preferred_element_type=F32)


def _dot_t(a, b):
    return lax.dot_general(a, b, (((1,), (1,)), ((), ())),
                           preferred_element_type=F32)


def _body(x_ref, wdkv_ref, wuk_ref, wuv_ref, wuk_g_ref, wuv_g_ref,
          wkr_ref, wq_ref, wqr_ref, wo_ref, out_ref,
          c_mine_s, c_oth_s, wuk_o, wuv_o, kr_s, out_acc,
          w_send, w_recv, copy_sem, ag_send, ag_recv):
    my_x = lax.axis_index("x")
    my_y = lax.axis_index("y")
    p = my_x * 2 + my_y
    xnbr = (1 - my_x, my_y)
    peers = [(1 - my_x, my_y), (my_x, 1 - my_y), (1 - my_x, 1 - my_y)]
    peer_slots = [(1 - my_x) * 2 + my_y, my_x * 2 + (1 - my_y),
                  (1 - my_x) * 2 + (1 - my_y)]
    g = pl.program_id(0)
    barrier = pltpu.get_barrier_semaphore()

    def _wrdma(i, src, dst):
        return pltpu.make_async_remote_copy(
            src_ref=src, dst_ref=dst,
            send_sem=w_send.at[i], recv_sem=w_recv.at[i],
            device_id=xnbr, device_id_type=pl.DeviceIdType.MESH)

    def _w_descs():
        descs = [_wrdma(0, c_mine_s, c_oth_s)]
        for gg in range(NG):
            descs.append(_wrdma(1 + 2 * gg,
                                wuk_ref.at[:, gg * GW:(gg + 1) * GW],
                                wuk_o.at[gg]))
            descs.append(_wrdma(2 + 2 * gg,
                                wuv_ref.at[:, gg * GW:(gg + 1) * GW],
                                wuv_o.at[gg]))
        return descs

    @pl.when(g == 0)
    def _():
        for peer in peers:
            pl.semaphore_signal(barrier, inc=1, device_id=peer,
                                device_id_type=pl.DeviceIdType.MESH)
        pl.semaphore_wait(barrier, 3)
        c_mine_s[...] = _dot(x_ref[0], wdkv_ref[...])
        for d in _w_descs():
            d.start()
        kr_s[...] = _dot(x_ref[0], wkr_ref[...])

    x_mine = x_ref[0, pl.ds(p * S4, S4), :]
    q = _dot(x_mine, wq_ref[...])
    qr = _dot(x_mine, wqr_ref[...])

    @pl.when(g == 0)
    def _():
        _wrdma(0, c_mine_s, c_oth_s).wait_recv()

    for gg in range(NG):
        @pl.when(g == gg)
        def _(gg=gg):
            _wrdma(1 + 2 * gg, wuk_ref.at[:, gg * GW:(gg + 1) * GW],
                   wuk_o.at[gg]).wait_recv()
            _wrdma(2 + 2 * gg, wuv_ref.at[:, gg * GW:(gg + 1) * GW],
                   wuv_o.at[gg]).wait_recv()

    c_mine = c_mine_s[...]
    c_oth = c_oth_s[...]
    k_g = _dot(c_mine, wuk_g_ref[...]) + _dot(c_oth, wuk_o[g])
    v_g = _dot(c_mine, wuv_g_ref[...]) + _dot(c_oth, wuv_o[g])
    kr = kr_s[...]

    o_cols = []
    for i in range(HG):
        s = _dot_t(q[:, i * DH:(i + 1) * DH], k_g[:, i * DH:(i + 1) * DH])
        s = s + _dot_t(qr[:, i * DR:(i + 1) * DR], kr)
        s = s * SCALE
        m = jnp.max(s, axis=1, keepdims=True)
        pr = jnp.exp(s - m)
        pr = pr / jnp.sum(pr, axis=1, keepdims=True)
        o_cols.append(_dot(pr, v_g[:, i * DH:(i + 1) * DH]))
    proj = _dot(jnp.concatenate(o_cols, axis=1), wo_ref[...])

    @pl.when(g == 0)
    def _():
        out_acc[...] = proj

    @pl.when(g != 0)
    def _():
        out_acc[...] = out_acc[...] + proj

    @pl.when(g == NG - 1)
    def _():
        my_rows = out_ref.at[0, pl.ds(p * S4, S4), :]
        cp = pltpu.make_async_copy(out_acc, my_rows, copy_sem)
        cp.start()
        cp.wait()
        sends = []
        for i, peer in enumerate(peers):
            r = pltpu.make_async_remote_copy(
                src_ref=my_rows, dst_ref=my_rows,
                send_sem=ag_send.at[i], recv_sem=ag_recv.at[p],
                device_id=peer, device_id_type=pl.DeviceIdType.MESH)
            r.start()
            sends.append(r)
        for d in _w_descs():
            d.wait_send()
        for r in sends:
            r.wait_send()
        for i, qp in enumerate(peer_slots):
            qrows = out_ref.at[0, pl.ds(qp * S4, S4), :]
            r = pltpu.make_async_remote_copy(
                src_ref=qrows, dst_ref=qrows,
                send_sem=ag_send.at[i], recv_sem=ag_recv.at[qp],
                device_id=(my_x, my_y), device_id_type=pl.DeviceIdType.MESH)
            r.wait_recv()


def kernel(x, Wdkv, Wuk, Wuv, Wq, Wqr, Wkr, Wo):
    y = pl.pallas_call(
        _body,
        grid=(NG,),
        out_shape=jax.ShapeDtypeStruct((1, S, D), F32),
        in_specs=[
            pl.BlockSpec((1, S, D), lambda g: (0, 0, 0)),
            pl.BlockSpec((D, DC_SH), lambda g: (0, 0)),
            pl.BlockSpec((DC_SH, D), lambda g: (0, 0)),
            pl.BlockSpec((DC_SH, D), lambda g: (0, 0)),
            pl.BlockSpec((DC_SH, GW), lambda g: (0, g)),
            pl.BlockSpec((DC_SH, GW), lambda g: (0, g)),
            pl.BlockSpec((D, DR), lambda g: (0, 0)),
            pl.BlockSpec((D, GW), lambda g: (0, g)),
            pl.BlockSpec((D, HG * DR), lambda g: (0, g)),
            pl.BlockSpec((GW, D), lambda g: (g, 0)),
        ],
        out_specs=pl.BlockSpec(memory_space=pl.ANY),
        scratch_shapes=[
            pltpu.VMEM((S, DC_SH), F32),
            pltpu.VMEM((S, DC_SH), F32),
            pltpu.VMEM((NG, DC_SH, GW), F32),
            pltpu.VMEM((NG, DC_SH, GW), F32),
            pltpu.VMEM((S, DR), F32),
            pltpu.VMEM((S4, D), F32),
            pltpu.SemaphoreType.DMA((9,)),
            pltpu.SemaphoreType.DMA((9,)),
            pltpu.SemaphoreType.DMA,
            pltpu.SemaphoreType.DMA((3,)),
            pltpu.SemaphoreType.DMA((4,)),
        ],
        compiler_params=pltpu.CompilerParams(
            collective_id=0, vmem_limit_bytes=60 * 1024 * 1024,
        ),
    )(x, Wdkv, Wuk, Wuv, Wuk, Wuv, Wkr, Wq, Wqr, Wo)

    return y
